# baseline (speedup 1.0000x reference)
import functools

import jax
import jax.numpy as jnp
import numpy as np
from jax.experimental import pallas as pl

INPUT_DIM = 768
META_DIM = 16
HID = 32
NUM_ACTIONS = 8
NUM_INPUTS = 2 * INPUT_DIM + META_DIM + 1
B = 16384
BLOCK_B = 1024


def _fused_body(x_ref, a_ref, w1_ref, b1_ref, w2_ref, b2_ref, w3_ref, b3_ref,
                out_ref):
    x = x_ref[...]
    h1 = jnp.tanh(
        jnp.dot(x, w1_ref[...], preferred_element_type=jnp.float32) + b1_ref[...])
    h2 = jnp.tanh(
        jnp.dot(h1, w2_ref[...], preferred_element_type=jnp.float32) + b2_ref[...])
    z = jnp.dot(h2, w3_ref[...], preferred_element_type=jnp.float32) + b3_ref[...]
    la = z[:, 0:NUM_ACTIONS]
    lb = z[:, NUM_ACTIONS:2 * NUM_ACTIONS]

    def _logp(logits):
        m = jnp.max(logits, axis=1, keepdims=True)
        lse = m + jnp.log(jnp.sum(jnp.exp(logits - m), axis=1, keepdims=True))
        idx = jax.lax.broadcasted_iota(jnp.int32, logits.shape, 1)
        oh = (idx == a_ref[...]).astype(jnp.float32)
        sel = jnp.sum(logits * oh, axis=1, keepdims=True)
        return sel - lse

    alogp = _logp(la)
    blogp = _logp(lb)
    amask = x[:, NUM_INPUTS - 1:NUM_INPUTS] == 1.0
    logp = jnp.where(amask, alogp, blogp)
    v = jnp.where(amask, z[:, 16:17], z[:, 17:18])
    out_ref[...] = jnp.concatenate([logp, v], axis=1)


@functools.partial(jax.jit, static_argnames=())
def kernel(x, a, aw1, ab1, aw2, ab2, aw3, ab3, bw1, bb1, bw2, bb2, bw3, bb3,
           acw1, acb1, acw2, acb2, acw3, acb3, bcw1, bcb1, bcw2, bcb2, bcw3,
           bcb3):
    n = INPUT_DIM + META_DIM
    f32 = jnp.float32

    w1a = jnp.zeros((NUM_INPUTS, HID), f32).at[:n].set(aw1)
    w1b = (jnp.zeros((NUM_INPUTS, HID), f32)
           .at[:INPUT_DIM].set(bw1[:INPUT_DIM])
           .at[n:n + INPUT_DIM].set(bw1[INPUT_DIM:]))
    w1 = jnp.concatenate([w1a, w1b, acw1, bcw1], axis=1)
    b1 = jnp.concatenate([ab1, bb1, acb1, bcb1])[None, :]

    zero = jnp.zeros((HID, HID), f32)
    w2 = jnp.block([[aw2, zero, zero, zero],
                    [zero, bw2, zero, zero],
                    [zero, zero, acw2, zero],
                    [zero, zero, zero, bcw2]])
    b2 = jnp.concatenate([ab2, bb2, acb2, bcb2])[None, :]

    w3 = jnp.zeros((4 * HID, 32), f32)
    w3 = w3.at[0:HID, 0:NUM_ACTIONS].set(aw3)
    w3 = w3.at[HID:2 * HID, NUM_ACTIONS:2 * NUM_ACTIONS].set(bw3)
    w3 = w3.at[2 * HID:3 * HID, 16:17].set(acw3)
    w3 = w3.at[3 * HID:4 * HID, 17:18].set(bcw3)
    b3 = jnp.zeros((1, 32), f32)
    b3 = b3.at[0, 0:NUM_ACTIONS].set(ab3)
    b3 = b3.at[0, NUM_ACTIONS:2 * NUM_ACTIONS].set(bb3)
    b3 = b3.at[0, 16].set(acb3[0])
    b3 = b3.at[0, 17].set(bcb3[0])

    a2 = a.astype(jnp.int32)[:, None]

    grid = (B // BLOCK_B,)
    out = pl.pallas_call(
        _fused_body,
        grid=grid,
        in_specs=[
            pl.BlockSpec((BLOCK_B, NUM_INPUTS), lambda i: (i, 0)),
            pl.BlockSpec((BLOCK_B, 1), lambda i: (i, 0)),
            pl.BlockSpec((NUM_INPUTS, 4 * HID), lambda i: (0, 0)),
            pl.BlockSpec((1, 4 * HID), lambda i: (0, 0)),
            pl.BlockSpec((4 * HID, 4 * HID), lambda i: (0, 0)),
            pl.BlockSpec((1, 4 * HID), lambda i: (0, 0)),
            pl.BlockSpec((4 * HID, 32), lambda i: (0, 0)),
            pl.BlockSpec((1, 32), lambda i: (0, 0)),
        ],
        out_specs=pl.BlockSpec((BLOCK_B, 2), lambda i: (i, 0)),
        out_shape=jax.ShapeDtypeStruct((B, 2), f32),
    )(x, a2, w1, b1, w2, b2, w3, b3)
    return out


# speedup vs baseline: 1.2638x; 1.2638x over previous
import jax
import jax.numpy as jnp
from jax.experimental import pallas as pl

INPUT_DIM = 768
META_DIM = 16
HID = 32
NUM_ACTIONS = 8
NUM_INPUTS = 2 * INPUT_DIM + META_DIM + 1
N_AC = INPUT_DIM + META_DIM
B = 16384
BLOCK_B = 512
MAIN_W = 1536
CHUNK_W = 384
TAIL_W = NUM_INPUTS - MAIN_W


def _assemble_body(aw1, ab1, aw2, ab2, aw3, ab3, bw1, bb1, bw2, bb2, bw3, bb3,
                   acw1, acb1, acw2, acb2, acw3, acb3, bcw1, bcb1, bcw2, bcb2,
                   bcw3, bcb3, w1o, b1o, w2o, b2o, w3o, b3o):
    f32 = jnp.float32
    w1o[0:N_AC, 0:HID] = aw1[...]
    w1o[N_AC:NUM_INPUTS, 0:HID] = jnp.zeros((NUM_INPUTS - N_AC, HID), f32)
    w1o[0:INPUT_DIM, HID:2 * HID] = bw1[0:INPUT_DIM, :]
    w1o[INPUT_DIM:N_AC, HID:2 * HID] = jnp.zeros((META_DIM, HID), f32)
    w1o[N_AC:N_AC + INPUT_DIM, HID:2 * HID] = bw1[INPUT_DIM:2 * INPUT_DIM, :]
    w1o[NUM_INPUTS - 1:NUM_INPUTS, HID:2 * HID] = jnp.zeros((1, HID), f32)
    w1o[:, 2 * HID:3 * HID] = acw1[...]
    w1o[:, 3 * HID:4 * HID] = bcw1[...]
    w2o[...] = jnp.zeros((4 * HID, 4 * HID), f32)
    w2o[0:HID, 0:HID] = aw2[...]
    w2o[HID:2 * HID, HID:2 * HID] = bw2[...]
    w2o[2 * HID:3 * HID, 2 * HID:3 * HID] = acw2[...]
    w2o[3 * HID:4 * HID, 3 * HID:4 * HID] = bcw2[...]
    w3o[...] = jnp.zeros((4 * HID, 32), f32)
    w3o[0:HID, 0:NUM_ACTIONS] = aw3[...]
    w3o[HID:2 * HID, NUM_ACTIONS:2 * NUM_ACTIONS] = bw3[...]
    w3o[2 * HID:3 * HID, 16:17] = acw3[...]
    w3o[3 * HID:4 * HID, 17:18] = bcw3[...]
    b1o[0:1, 0:HID] = ab1[...].reshape(1, HID)
    b1o[0:1, HID:2 * HID] = bb1[...].reshape(1, HID)
    b1o[0:1, 2 * HID:3 * HID] = acb1[...].reshape(1, HID)
    b1o[0:1, 3 * HID:4 * HID] = bcb1[...].reshape(1, HID)
    b2o[0:1, 0:HID] = ab2[...].reshape(1, HID)
    b2o[0:1, HID:2 * HID] = bb2[...].reshape(1, HID)
    b2o[0:1, 2 * HID:3 * HID] = acb2[...].reshape(1, HID)
    b2o[0:1, 3 * HID:4 * HID] = bcb2[...].reshape(1, HID)
    b3o[...] = jnp.zeros((1, 32), f32)
    b3o[0:1, 0:NUM_ACTIONS] = ab3[...].reshape(1, NUM_ACTIONS)
    b3o[0:1, NUM_ACTIONS:2 * NUM_ACTIONS] = bb3[...].reshape(1, NUM_ACTIONS)
    b3o[0:1, 16:17] = acb3[...].reshape(1, 1)
    b3o[0:1, 17:18] = bcb3[...].reshape(1, 1)


def _main_body(x0, x1, x2, x3, xt, a_ref, w1, b1, w2, b2, w3, b3, out_ref):
    acc = jnp.dot(x0[...], w1[0 * CHUNK_W:1 * CHUNK_W, :],
                  preferred_element_type=jnp.float32)
    acc += jnp.dot(x1[...], w1[1 * CHUNK_W:2 * CHUNK_W, :],
                   preferred_element_type=jnp.float32)
    acc += jnp.dot(x2[...], w1[2 * CHUNK_W:3 * CHUNK_W, :],
                   preferred_element_type=jnp.float32)
    acc += jnp.dot(x3[...], w1[3 * CHUNK_W:4 * CHUNK_W, :],
                   preferred_element_type=jnp.float32)
    tail = xt[...]
    acc += jnp.dot(tail, w1[MAIN_W:NUM_INPUTS, :],
                   preferred_element_type=jnp.float32)
    h1 = jnp.tanh(acc + b1[...])
    h2 = jnp.tanh(
        jnp.dot(h1, w2[...], preferred_element_type=jnp.float32) + b2[...])
    z = jnp.dot(h2, w3[...], preferred_element_type=jnp.float32) + b3[...]

    amask = tail[:, TAIL_W - 1:TAIL_W] == 1.0
    logits = jnp.where(amask, z[:, 0:NUM_ACTIONS],
                       z[:, NUM_ACTIONS:2 * NUM_ACTIONS])
    m = jnp.max(logits, axis=1, keepdims=True)
    lse = m + jnp.log(jnp.sum(jnp.exp(logits - m), axis=1, keepdims=True))
    idx = jax.lax.broadcasted_iota(jnp.int32, logits.shape, 1)
    oh = (idx == a_ref[...]).astype(jnp.float32)
    sel = jnp.sum(logits * oh, axis=1, keepdims=True)
    logp = sel - lse
    v = jnp.where(amask, z[:, 16:17], z[:, 17:18])
    out_ref[...] = jnp.concatenate([logp, v], axis=1)


def kernel(x, a, aw1, ab1, aw2, ab2, aw3, ab3, bw1, bb1, bw2, bb2, bw3, bb3,
           acw1, acb1, acw2, acb2, acw3, acb3, bcw1, bcb1, bcw2, bcb2, bcw3,
           bcb3):
    f32 = jnp.float32
    full = lambda s: pl.BlockSpec(s, lambda: (0,) * len(s))
    w1, b1, w2, b2, w3, b3 = pl.pallas_call(
        _assemble_body,
        in_specs=[full(t.shape) for t in (
            aw1, ab1, aw2, ab2, aw3, ab3, bw1, bb1, bw2, bb2, bw3, bb3,
            acw1, acb1, acw2, acb2, acw3, acb3, bcw1, bcb1, bcw2, bcb2,
            bcw3, bcb3)],
        out_specs=[full((NUM_INPUTS, 4 * HID)), full((1, 4 * HID)),
                   full((4 * HID, 4 * HID)), full((1, 4 * HID)),
                   full((4 * HID, 32)), full((1, 32))],
        out_shape=[jax.ShapeDtypeStruct((NUM_INPUTS, 4 * HID), f32),
                   jax.ShapeDtypeStruct((1, 4 * HID), f32),
                   jax.ShapeDtypeStruct((4 * HID, 4 * HID), f32),
                   jax.ShapeDtypeStruct((1, 4 * HID), f32),
                   jax.ShapeDtypeStruct((4 * HID, 32), f32),
                   jax.ShapeDtypeStruct((1, 32), f32)],
    )(aw1, ab1, aw2, ab2, aw3, ab3, bw1, bb1, bw2, bb2, bw3, bb3,
      acw1, acb1, acw2, acb2, acw3, acb3, bcw1, bcb1, bcw2, bcb2, bcw3, bcb3)

    xt = jax.lax.slice(x, (0, MAIN_W), (B, NUM_INPUTS))
    a2 = a.astype(jnp.int32)[:, None]

    grid = (B // BLOCK_B,)
    chunk = lambda j: pl.BlockSpec((BLOCK_B, CHUNK_W), lambda i: (i, j))
    out = pl.pallas_call(
        _main_body,
        grid=grid,
        in_specs=[
            chunk(0), chunk(1), chunk(2), chunk(3),
            pl.BlockSpec((BLOCK_B, TAIL_W), lambda i: (i, 0)),
            pl.BlockSpec((BLOCK_B, 1), lambda i: (i, 0)),
            pl.BlockSpec((NUM_INPUTS, 4 * HID), lambda i: (0, 0)),
            pl.BlockSpec((1, 4 * HID), lambda i: (0, 0)),
            pl.BlockSpec((4 * HID, 4 * HID), lambda i: (0, 0)),
            pl.BlockSpec((1, 4 * HID), lambda i: (0, 0)),
            pl.BlockSpec((4 * HID, 32), lambda i: (0, 0)),
            pl.BlockSpec((1, 32), lambda i: (0, 0)),
        ],
        out_specs=pl.BlockSpec((BLOCK_B, 2), lambda i: (i, 0)),
        out_shape=jax.ShapeDtypeStruct((B, 2), f32),
    )(x, x, x, x, xt, a2, w1, b1, w2, b2, w3, b3)
    return out


# speedup vs baseline: 1.3931x; 1.1023x over previous
import jax
import jax.numpy as jnp
from jax.experimental import pallas as pl

INPUT_DIM = 768
META_DIM = 16
HID = 32
NUM_ACTIONS = 8
NUM_INPUTS = 2 * INPUT_DIM + META_DIM + 1
N_AC = INPUT_DIM + META_DIM
B = 16384
SUB_B = 512
N_STREAMS = 4
STEP_B = SUB_B * N_STREAMS
MAIN_W = 1536
CHUNK_W = 384
TAIL_W = NUM_INPUTS - MAIN_W


def _assemble_body(aw1, ab1, aw2, ab2, aw3, ab3, bw1, bb1, bw2, bb2, bw3, bb3,
                   acw1, acb1, acw2, acb2, acw3, acb3, bcw1, bcb1, bcw2, bcb2,
                   bcw3, bcb3, w1o, b1o, w2o, b2o, w3o, b3o):
    f32 = jnp.float32
    w1o[0:N_AC, 0:HID] = aw1[...]
    w1o[N_AC:NUM_INPUTS, 0:HID] = jnp.zeros((NUM_INPUTS - N_AC, HID), f32)
    w1o[0:INPUT_DIM, HID:2 * HID] = bw1[0:INPUT_DIM, :]
    w1o[INPUT_DIM:N_AC, HID:2 * HID] = jnp.zeros((META_DIM, HID), f32)
    w1o[N_AC:N_AC + INPUT_DIM, HID:2 * HID] = bw1[INPUT_DIM:2 * INPUT_DIM, :]
    w1o[NUM_INPUTS - 1:NUM_INPUTS, HID:2 * HID] = jnp.zeros((1, HID), f32)
    w1o[:, 2 * HID:3 * HID] = acw1[...]
    w1o[:, 3 * HID:4 * HID] = bcw1[...]
    w2o[...] = jnp.zeros((4 * HID, 4 * HID), f32)
    w2o[0:HID, 0:HID] = aw2[...]
    w2o[HID:2 * HID, HID:2 * HID] = bw2[...]
    w2o[2 * HID:3 * HID, 2 * HID:3 * HID] = acw2[...]
    w2o[3 * HID:4 * HID, 3 * HID:4 * HID] = bcw2[...]
    w3o[...] = jnp.zeros((4 * HID, 32), f32)
    w3o[0:HID, 0:NUM_ACTIONS] = aw3[...]
    w3o[HID:2 * HID, NUM_ACTIONS:2 * NUM_ACTIONS] = bw3[...]
    w3o[2 * HID:3 * HID, 16:17] = acw3[...]
    w3o[3 * HID:4 * HID, 17:18] = bcw3[...]
    b1o[0:1, 0:HID] = ab1[...].reshape(1, HID)
    b1o[0:1, HID:2 * HID] = bb1[...].reshape(1, HID)
    b1o[0:1, 2 * HID:3 * HID] = acb1[...].reshape(1, HID)
    b1o[0:1, 3 * HID:4 * HID] = bcb1[...].reshape(1, HID)
    b2o[0:1, 0:HID] = ab2[...].reshape(1, HID)
    b2o[0:1, HID:2 * HID] = bb2[...].reshape(1, HID)
    b2o[0:1, 2 * HID:3 * HID] = acb2[...].reshape(1, HID)
    b2o[0:1, 3 * HID:4 * HID] = bcb2[...].reshape(1, HID)
    b3o[...] = jnp.zeros((1, 32), f32)
    b3o[0:1, 0:NUM_ACTIONS] = ab3[...].reshape(1, NUM_ACTIONS)
    b3o[0:1, NUM_ACTIONS:2 * NUM_ACTIONS] = bb3[...].reshape(1, NUM_ACTIONS)
    b3o[0:1, 16:17] = acb3[...].reshape(1, 1)
    b3o[0:1, 17:18] = bcb3[...].reshape(1, 1)


def _main_body(x0, x1, x2, x3, a_ref, w1, b1, w2, b2, w3, b3, out_ref):
    for k, x_ref in enumerate((x0, x1, x2, x3)):
        acc = jnp.dot(x_ref[:, 0 * CHUNK_W:1 * CHUNK_W],
                      w1[0 * CHUNK_W:1 * CHUNK_W, :],
                      preferred_element_type=jnp.float32)
        acc += jnp.dot(x_ref[:, 1 * CHUNK_W:2 * CHUNK_W],
                       w1[1 * CHUNK_W:2 * CHUNK_W, :],
                       preferred_element_type=jnp.float32)
        acc += jnp.dot(x_ref[:, 2 * CHUNK_W:3 * CHUNK_W],
                       w1[2 * CHUNK_W:3 * CHUNK_W, :],
                       preferred_element_type=jnp.float32)
        acc += jnp.dot(x_ref[:, 3 * CHUNK_W:4 * CHUNK_W],
                       w1[3 * CHUNK_W:4 * CHUNK_W, :],
                       preferred_element_type=jnp.float32)
        tail = x_ref[:, MAIN_W:NUM_INPUTS]
        acc += jnp.dot(tail, w1[MAIN_W:NUM_INPUTS, :],
                       preferred_element_type=jnp.float32)
        h1 = jnp.tanh(acc + b1[...])
        h2 = jnp.tanh(
            jnp.dot(h1, w2[...], preferred_element_type=jnp.float32) + b2[...])
        z = jnp.dot(h2, w3[...], preferred_element_type=jnp.float32) + b3[...]

        amask = tail[:, TAIL_W - 1:TAIL_W] == 1.0
        logits = jnp.where(amask, z[:, 0:NUM_ACTIONS],
                           z[:, NUM_ACTIONS:2 * NUM_ACTIONS])
        m = jnp.max(logits, axis=1, keepdims=True)
        lse = m + jnp.log(jnp.sum(jnp.exp(logits - m), axis=1, keepdims=True))
        idx = jax.lax.broadcasted_iota(jnp.int32, logits.shape, 1)
        oh = (idx == a_ref[k * SUB_B:(k + 1) * SUB_B, :]).astype(jnp.float32)
        sel = jnp.sum(logits * oh, axis=1, keepdims=True)
        logp = sel - lse
        v = jnp.where(amask, z[:, 16:17], z[:, 17:18])
        out_ref[k * SUB_B:(k + 1) * SUB_B, :] = jnp.concatenate([logp, v],
                                                                axis=1)


def kernel(x, a, aw1, ab1, aw2, ab2, aw3, ab3, bw1, bb1, bw2, bb2, bw3, bb3,
           acw1, acb1, acw2, acb2, acw3, acb3, bcw1, bcb1, bcw2, bcb2, bcw3,
           bcb3):
    f32 = jnp.float32
    full = lambda s: pl.BlockSpec(s, lambda: (0,) * len(s))
    w1, b1, w2, b2, w3, b3 = pl.pallas_call(
        _assemble_body,
        in_specs=[full(t.shape) for t in (
            aw1, ab1, aw2, ab2, aw3, ab3, bw1, bb1, bw2, bb2, bw3, bb3,
            acw1, acb1, acw2, acb2, acw3, acb3, bcw1, bcb1, bcw2, bcb2,
            bcw3, bcb3)],
        out_specs=[full((NUM_INPUTS, 4 * HID)), full((1, 4 * HID)),
                   full((4 * HID, 4 * HID)), full((1, 4 * HID)),
                   full((4 * HID, 32)), full((1, 32))],
        out_shape=[jax.ShapeDtypeStruct((NUM_INPUTS, 4 * HID), f32),
                   jax.ShapeDtypeStruct((1, 4 * HID), f32),
                   jax.ShapeDtypeStruct((4 * HID, 4 * HID), f32),
                   jax.ShapeDtypeStruct((1, 4 * HID), f32),
                   jax.ShapeDtypeStruct((4 * HID, 32), f32),
                   jax.ShapeDtypeStruct((1, 32), f32)],
    )(aw1, ab1, aw2, ab2, aw3, ab3, bw1, bb1, bw2, bb2, bw3, bb3,
      acw1, acb1, acw2, acb2, acw3, acb3, bcw1, bcb1, bcw2, bcb2, bcw3, bcb3)

    a2 = a.astype(jnp.int32)[:, None]

    grid = (B // STEP_B,)
    xs = lambda k: pl.BlockSpec((SUB_B, NUM_INPUTS),
                                lambda i, kk=k: (N_STREAMS * i + kk, 0))
    out = pl.pallas_call(
        _main_body,
        grid=grid,
        in_specs=[
            xs(0), xs(1), xs(2), xs(3),
            pl.BlockSpec((STEP_B, 1), lambda i: (i, 0)),
            pl.BlockSpec((NUM_INPUTS, 4 * HID), lambda i: (0, 0)),
            pl.BlockSpec((1, 4 * HID), lambda i: (0, 0)),
            pl.BlockSpec((4 * HID, 4 * HID), lambda i: (0, 0)),
            pl.BlockSpec((1, 4 * HID), lambda i: (0, 0)),
            pl.BlockSpec((4 * HID, 32), lambda i: (0, 0)),
            pl.BlockSpec((1, 32), lambda i: (0, 0)),
        ],
        out_specs=pl.BlockSpec((STEP_B, 2), lambda i: (i, 0)),
        out_shape=jax.ShapeDtypeStruct((B, 2), f32),
    )(x, x, x, x, a2, w1, b1, w2, b2, w3, b3)
    return out
